# baseline (speedup 1.0000x reference)
import functools

import jax
import jax.numpy as jnp
from jax import lax
from jax.experimental import pallas as pl
from jax.experimental.pallas import tpu as pltpu

_DOT_LAST_LAST = (((1,), (1,)), ((), ()))

_VMEM_LIMIT = 60 * 1024 * 1024


def _round_up(v, m):
    return (v + m - 1) // m * m


def _body(x_ref, w_ref, m_ref, o_ref, wg_ref, *, tn):
    j = pl.program_id(1)

    @pl.when(pl.program_id(0) == 0)
    def _():
        wg_ref[pl.ds(j * tn, tn), :] = (
            jnp.tanh(w_ref[...]) * jax.nn.sigmoid(m_ref[...]))

    o_ref[...] = lax.dot_general(
        x_ref[...], wg_ref[pl.ds(j * tn, tn), :],
        dimension_numbers=_DOT_LAST_LAST,
        preferred_element_type=jnp.float32,
        precision=lax.Precision.DEFAULT,
    )


def _nac_fused(x, w_, m_, tm):
    B, K = x.shape
    N = w_.shape[0]
    tm = min(tm, _round_up(B, 8))
    Bp = _round_up(B, tm)
    if Bp != B:
        x = jnp.pad(x, ((0, Bp - B), (0, 0)))

    nj = 2 if N % 256 == 0 else 1
    tn = N // nj
    wslice = pl.BlockSpec((tn, K), lambda i, j: (j, 0))
    yp = pl.pallas_call(
        functools.partial(_body, tn=tn),
        out_shape=jax.ShapeDtypeStruct((Bp, N), jnp.float32),
        grid=(Bp // tm, nj),
        in_specs=[
            pl.BlockSpec((tm, K), lambda i, j: (i, 0)),
            wslice,
            wslice,
        ],
        out_specs=pl.BlockSpec((tm, tn), lambda i, j: (i, j)),
        scratch_shapes=[pltpu.VMEM((N, K), jnp.float32)],
        compiler_params=pltpu.CompilerParams(
            dimension_semantics=("arbitrary", "arbitrary"),
            vmem_limit_bytes=_VMEM_LIMIT,
        ),
    )(x, w_, m_)
    return yp[:B] if Bp != B else yp


def kernel(x, w_, m_):
    assert x.ndim == 2 and w_.shape == m_.shape and x.shape[1] == w_.shape[1]
    return _nac_fused(x, w_, m_, tm=2048)


# speedup vs baseline: 1.3187x; 1.3187x over previous
import functools

import jax
import jax.numpy as jnp
from jax import lax
from jax.experimental import pallas as pl
from jax.experimental.pallas import tpu as pltpu

_DOT_LAST_LAST = (((1,), (1,)), ((), ()))

_VMEM_LIMIT = 60 * 1024 * 1024


def _round_up(v, m):
    return (v + m - 1) // m * m


def _body(x_ref, w_ref, m_ref, o_ref, wg_ref):
    @pl.when(pl.program_id(0) == 0)
    def _():
        wg_ref[...] = jnp.tanh(w_ref[...]) * jax.nn.sigmoid(m_ref[...])

    o_ref[...] = lax.dot_general(
        x_ref[...], wg_ref[...],
        dimension_numbers=_DOT_LAST_LAST,
        preferred_element_type=jnp.float32,
        precision=lax.Precision.DEFAULT,
    )


def _nac_fused(x, w_, m_, tm):
    B, K = x.shape
    N = w_.shape[0]
    tm = min(tm, _round_up(B, 8))
    Bp = _round_up(B, tm)
    if Bp != B:
        x = jnp.pad(x, ((0, Bp - B), (0, 0)))

    wfull = pl.BlockSpec((N, K), lambda i: (0, 0))
    yp = pl.pallas_call(
        _body,
        out_shape=jax.ShapeDtypeStruct((Bp, N), jnp.float32),
        grid=(Bp // tm,),
        in_specs=[
            pl.BlockSpec((tm, K), lambda i: (i, 0)),
            wfull,
            wfull,
        ],
        out_specs=pl.BlockSpec((tm, N), lambda i: (i, 0)),
        scratch_shapes=[pltpu.VMEM((N, K), jnp.float32)],
        compiler_params=pltpu.CompilerParams(
            dimension_semantics=("arbitrary",),
            vmem_limit_bytes=_VMEM_LIMIT,
        ),
    )(x, w_, m_)
    return yp[:B] if Bp != B else yp


def kernel(x, w_, m_):
    assert x.ndim == 2 and w_.shape == m_.shape and x.shape[1] == w_.shape[1]
    return _nac_fused(x, w_, m_, tm=1024)


# speedup vs baseline: 1.3556x; 1.0280x over previous
import functools

import jax
import jax.numpy as jnp
from jax import lax
from jax.experimental import pallas as pl
from jax.experimental.pallas import tpu as pltpu

_DOT_LAST_LAST = (((1,), (1,)), ((), ()))

_VMEM_LIMIT = 60 * 1024 * 1024


def _round_up(v, m):
    return (v + m - 1) // m * m


def _body(x_ref, w_ref, m_ref, o_ref, wg_ref):
    @pl.when(pl.program_id(0) == 0)
    def _():
        wg_ref[...] = jnp.tanh(w_ref[...]) * jax.nn.sigmoid(m_ref[...])

    o_ref[...] = lax.dot_general(
        x_ref[...], wg_ref[...],
        dimension_numbers=_DOT_LAST_LAST,
        preferred_element_type=jnp.float32,
        precision=lax.Precision.DEFAULT,
    )


def _nac_fused(x, w_, m_, tm):
    B, K = x.shape
    N = w_.shape[0]
    tm = min(tm, _round_up(B, 8))
    Bp = _round_up(B, tm)
    if Bp != B:
        x = jnp.pad(x, ((0, Bp - B), (0, 0)))

    wfull = pl.BlockSpec((N, K), lambda i: (0, 0))
    yp = pl.pallas_call(
        _body,
        out_shape=jax.ShapeDtypeStruct((Bp, N), jnp.float32),
        grid=(Bp // tm,),
        in_specs=[
            pl.BlockSpec((tm, K), lambda i: (i, 0)),
            wfull,
            wfull,
        ],
        out_specs=pl.BlockSpec((tm, N), lambda i: (i, 0)),
        scratch_shapes=[pltpu.VMEM((N, K), jnp.float32)],
        compiler_params=pltpu.CompilerParams(
            dimension_semantics=("arbitrary",),
            vmem_limit_bytes=_VMEM_LIMIT,
        ),
    )(x, w_, m_)
    return yp[:B] if Bp != B else yp


def kernel(x, w_, m_):
    assert x.ndim == 2 and w_.shape == m_.shape and x.shape[1] == w_.shape[1]
    return _nac_fused(x, w_, m_, tm=2048)


# speedup vs baseline: 1.3700x; 1.0106x over previous
import functools

import jax
import jax.numpy as jnp
from jax import lax
from jax.experimental import pallas as pl
from jax.experimental.pallas import tpu as pltpu

_DOT_LAST_LAST = (((1,), (1,)), ((), ()))

_VMEM_LIMIT = 60 * 1024 * 1024


def _round_up(v, m):
    return (v + m - 1) // m * m


def _body(x_ref, w_ref, m_ref, o_ref, wg_ref):
    @pl.when(pl.program_id(0) == 0)
    def _():
        sig = 0.5 + 0.5 * jnp.tanh(0.5 * m_ref[...])
        wg_ref[...] = jnp.tanh(w_ref[...]) * sig

    o_ref[...] = lax.dot_general(
        x_ref[...], wg_ref[...],
        dimension_numbers=_DOT_LAST_LAST,
        preferred_element_type=jnp.float32,
        precision=lax.Precision.DEFAULT,
    )


def _nac_fused(x, w_, m_, tm):
    B, K = x.shape
    N = w_.shape[0]
    tm = min(tm, _round_up(B, 8))
    Bp = _round_up(B, tm)
    if Bp != B:
        x = jnp.pad(x, ((0, Bp - B), (0, 0)))

    wfull = pl.BlockSpec((N, K), lambda i: (0, 0))
    yp = pl.pallas_call(
        _body,
        out_shape=jax.ShapeDtypeStruct((Bp, N), jnp.float32),
        grid=(Bp // tm,),
        in_specs=[
            pl.BlockSpec((tm, K), lambda i: (i, 0)),
            wfull,
            wfull,
        ],
        out_specs=pl.BlockSpec((tm, N), lambda i: (i, 0)),
        scratch_shapes=[pltpu.VMEM((N, K), jnp.float32)],
        compiler_params=pltpu.CompilerParams(
            dimension_semantics=("arbitrary",),
            vmem_limit_bytes=_VMEM_LIMIT,
        ),
    )(x, w_, m_)
    return yp[:B] if Bp != B else yp


def kernel(x, w_, m_):
    assert x.ndim == 2 and w_.shape == m_.shape and x.shape[1] == w_.shape[1]
    return _nac_fused(x, w_, m_, tm=2048)


# speedup vs baseline: 1.3710x; 1.0008x over previous
import functools

import jax
import jax.numpy as jnp
from jax import lax
from jax.experimental import pallas as pl
from jax.experimental.pallas import tpu as pltpu

_DOT_LAST_LAST = (((1,), (1,)), ((), ()))

_VMEM_LIMIT = 60 * 1024 * 1024


def _round_up(v, m):
    return (v + m - 1) // m * m


def _body(x_ref, w_ref, m_ref, o_ref, wg_ref):
    @pl.when(pl.program_id(0) == 0)
    def _():
        sig = 0.5 + 0.5 * jnp.tanh(0.5 * m_ref[...])
        wg_ref[...] = (jnp.tanh(w_ref[...]) * sig).astype(jnp.bfloat16)

    o_ref[...] = lax.dot_general(
        x_ref[...].astype(jnp.bfloat16), wg_ref[...],
        dimension_numbers=_DOT_LAST_LAST,
        preferred_element_type=jnp.float32,
        precision=lax.Precision.DEFAULT,
    )


def _nac_fused(x, w_, m_, tm):
    B, K = x.shape
    N = w_.shape[0]
    tm = min(tm, _round_up(B, 8))
    Bp = _round_up(B, tm)
    if Bp != B:
        x = jnp.pad(x, ((0, Bp - B), (0, 0)))

    wfull = pl.BlockSpec((N, K), lambda i: (0, 0))
    yp = pl.pallas_call(
        _body,
        out_shape=jax.ShapeDtypeStruct((Bp, N), jnp.float32),
        grid=(Bp // tm,),
        in_specs=[
            pl.BlockSpec((tm, K), lambda i: (i, 0)),
            wfull,
            wfull,
        ],
        out_specs=pl.BlockSpec((tm, N), lambda i: (i, 0)),
        scratch_shapes=[pltpu.VMEM((N, K), jnp.bfloat16)],
        compiler_params=pltpu.CompilerParams(
            dimension_semantics=("arbitrary",),
            vmem_limit_bytes=_VMEM_LIMIT,
        ),
    )(x, w_, m_)
    return yp[:B] if Bp != B else yp


def kernel(x, w_, m_):
    assert x.ndim == 2 and w_.shape == m_.shape and x.shape[1] == w_.shape[1]
    return _nac_fused(x, w_, m_, tm=2048)


# speedup vs baseline: 1.4105x; 1.0288x over previous
import functools

import jax
import jax.numpy as jnp
from jax import lax
from jax.experimental import pallas as pl
from jax.experimental.pallas import tpu as pltpu

_DOT_LAST_LAST = (((1,), (1,)), ((), ()))

_VMEM_LIMIT = 60 * 1024 * 1024


def _round_up(v, m):
    return (v + m - 1) // m * m


def _body(x_hbm, w_hbm, m_hbm, y_hbm,
          wb_ref, mb_ref, wg_ref, xb_ref, yb_ref,
          wm_sem, in_sem, out_sem, *, tm, n_steps):
    def dma_in(slot, step):
        pltpu.make_async_copy(
            x_hbm.at[pl.ds(step * tm, tm), :], xb_ref.at[slot],
            in_sem.at[slot]).start()

    def wait_in(slot):
        pltpu.make_async_copy(
            x_hbm.at[pl.ds(0, tm), :], xb_ref.at[slot],
            in_sem.at[slot]).wait()

    def dma_out(slot, step):
        pltpu.make_async_copy(
            yb_ref.at[slot], y_hbm.at[pl.ds(step * tm, tm), :],
            out_sem.at[slot]).start()

    def wait_out(slot):
        pltpu.make_async_copy(
            yb_ref.at[slot], y_hbm.at[pl.ds(0, tm), :],
            out_sem.at[slot]).wait()

    pltpu.make_async_copy(w_hbm, wb_ref, wm_sem.at[0]).start()
    pltpu.make_async_copy(m_hbm, mb_ref, wm_sem.at[1]).start()
    dma_in(0, 0)

    pltpu.make_async_copy(w_hbm, wb_ref, wm_sem.at[0]).wait()
    pltpu.make_async_copy(m_hbm, mb_ref, wm_sem.at[1]).wait()
    wg_ref[...] = jnp.tanh(wb_ref[...]) * (
        0.5 + 0.5 * jnp.tanh(0.5 * mb_ref[...]))

    def step_fn(step, _):
        cur = lax.rem(step, 2)
        nxt = lax.rem(step + 1, 2)

        @pl.when(step + 1 < n_steps)
        def _():
            dma_in(nxt, step + 1)

        wait_in(cur)

        @pl.when(step >= 2)
        def _():
            wait_out(cur)

        yb_ref[cur] = lax.dot_general(
            xb_ref[cur], wg_ref[...],
            dimension_numbers=_DOT_LAST_LAST,
            preferred_element_type=jnp.float32,
            precision=lax.Precision.DEFAULT,
        )
        dma_out(cur, step)
        return ()

    lax.fori_loop(0, n_steps, step_fn, (), unroll=False)

    if n_steps > 1:
        wait_out((n_steps - 2) % 2)
    wait_out((n_steps - 1) % 2)


def _nac_manual(x, w_, m_, tm):
    B, K = x.shape
    N = w_.shape[0]
    tm = min(tm, _round_up(B, 8))
    Bp = _round_up(B, tm)
    if Bp != B:
        x = jnp.pad(x, ((0, Bp - B), (0, 0)))
    n_steps = Bp // tm

    anyspec = pl.BlockSpec(memory_space=pltpu.MemorySpace.HBM)
    yp = pl.pallas_call(
        functools.partial(_body, tm=tm, n_steps=n_steps),
        out_shape=jax.ShapeDtypeStruct((Bp, N), jnp.float32),
        in_specs=[anyspec, anyspec, anyspec],
        out_specs=anyspec,
        scratch_shapes=[
            pltpu.VMEM((N, K), jnp.float32),
            pltpu.VMEM((N, K), jnp.float32),
            pltpu.VMEM((N, K), jnp.float32),
            pltpu.VMEM((2, tm, K), jnp.float32),
            pltpu.VMEM((2, tm, N), jnp.float32),
            pltpu.SemaphoreType.DMA((2,)),
            pltpu.SemaphoreType.DMA((2,)),
            pltpu.SemaphoreType.DMA((2,)),
        ],
        compiler_params=pltpu.CompilerParams(
            vmem_limit_bytes=_VMEM_LIMIT,
        ),
    )(x, w_, m_)
    return yp[:B] if Bp != B else yp


def kernel(x, w_, m_):
    assert x.ndim == 2 and w_.shape == m_.shape and x.shape[1] == w_.shape[1]
    return _nac_manual(x, w_, m_, tm=2048)
